# baseline (speedup 1.0000x reference)
import functools

import jax
import jax.numpy as jnp
from jax import lax
from jax.experimental import pallas as pl
from jax.experimental.pallas import tpu as pltpu
from jax.experimental.pallas import tpu_sc as plsc

H = 768
E = 8
FF = 768
T = 2048
BLK = 256
PAD_T = T * 2 + E * BLK
NB = PAD_T // BLK


def _routing_body(x_ref, wg_ref, s0_ref, s1_ref, g0_ref, g1_ref,
                  w0_ref, w1_ref, be_ref, st_ref):
    x = x_ref[...]
    wg = wg_ref[...]
    logits = jnp.dot(x, wg, preferred_element_type=jnp.float32)
    eidx = lax.broadcasted_iota(jnp.int32, (T, E), 1)
    m0 = jnp.max(logits, axis=1, keepdims=True)
    sel0 = jnp.min(jnp.where(logits == m0, eidx, E), axis=1, keepdims=True)
    masked = jnp.where(eidx == sel0, -jnp.inf, logits)
    m1 = jnp.max(masked, axis=1, keepdims=True)
    sel1 = jnp.min(jnp.where(masked == m1, eidx, E), axis=1, keepdims=True)
    ed = jnp.exp(m1 - m0)
    w1_ref[...] = ed / (1.0 + ed)
    w0_ref[...] = 1.0 / (1.0 + ed)

    p = jnp.exp(logits - m0)
    probs = p / jnp.sum(p, axis=1, keepdims=True)
    usage = jnp.mean(probs, axis=0, keepdims=True)
    um = jnp.mean(usage)
    var = jnp.mean((usage - um) ** 2)
    balance = var * float(E)
    lane = lax.broadcasted_iota(jnp.int32, (1, E), 1)
    rsum = jnp.float32(0.0)
    for i in range(E):
        u_i = usage[0, i]
        less_i = jnp.sum(jnp.where(usage < u_i, 1.0, 0.0))
        tie_i = jnp.sum(jnp.where((usage == u_i) & (lane < i), 1.0, 0.0))
        rsum = rsum + u_i * (1.0 + less_i + tie_i)
    usum = jnp.sum(usage)
    gini = 2.0 * rsum / (float(E) * usum) - float(E + 1) / float(E)
    ent = -jnp.mean(jnp.sum(probs * jnp.log(probs + 1e-10), axis=1))

    oh0 = jnp.where(eidx == sel0, 1.0, 0.0)
    oh1 = jnp.where(eidx == sel1, 1.0, 0.0)
    oha = oh0 + oh1
    cb = 128
    r_i = lax.broadcasted_iota(jnp.int32, (cb, cb), 0)
    c_i = lax.broadcasted_iota(jnp.int32, (cb, cb), 1)
    ltri = jnp.where(r_i > c_i, 1.0, 0.0)

    def ex_cumsum(oh):
        run = jnp.zeros((1, E), jnp.float32)
        outs = []
        for b in range(T // cb):
            blk = lax.slice(oh, (b * cb, 0), ((b + 1) * cb, E))
            outs.append(run + jnp.dot(ltri, blk,
                                      preferred_element_type=jnp.float32))
            run = run + jnp.sum(blk, axis=0, keepdims=True)
        return jnp.concatenate(outs, axis=0), run

    rank0, _ = ex_cumsum(oh0)
    rank1, _ = ex_cumsum(oh1)
    ranka, ca = ex_cumsum(oha)
    padc = jnp.floor((ca + float(BLK - 1)) / float(BLK)) * float(BLK)
    ii = lax.broadcasted_iota(jnp.int32, (E, E), 0)
    jj = lax.broadcasted_iota(jnp.int32, (E, E), 1)
    utri = jnp.where(ii < jj, 1.0, 0.0)
    off = jnp.dot(padc, utri, preferred_element_type=jnp.float32)
    s0_ref[...] = jnp.sum(oh0 * (ranka + off), axis=1, keepdims=True)
    s1_ref[...] = jnp.sum(oh1 * (ranka + off), axis=1, keepdims=True)
    g0_ref[...] = jnp.sum(oh0 * (rank0 + off), axis=1, keepdims=True)
    g1_ref[...] = jnp.sum(oh1 * (rank1 + off), axis=1, keepdims=True)

    bvals = lax.broadcasted_iota(jnp.int32, (NB, E), 0).astype(jnp.float32) * float(BLK)
    offnb = jnp.broadcast_to(off, (NB, E))
    cnt = jnp.sum(jnp.where(offnb <= bvals, 1.0, 0.0), axis=1, keepdims=True)
    be_ref[...] = jnp.clip(cnt - 1.0, 0.0, float(E - 1))

    sr = lax.broadcasted_iota(jnp.int32, (8, 128), 0)
    sc = lax.broadcasted_iota(jnp.int32, (8, 128), 1)
    st = jnp.where((sr == 0) & (sc == 0), balance, 0.0)
    st = st + jnp.where((sr == 0) & (sc == 1), var, 0.0)
    st = st + jnp.where((sr == 0) & (sc == 2), gini, 0.0)
    st_ref[...] = st + jnp.where((sr == 0) & (sc == 3), ent, 0.0)


def _mlp_body(be_ref, x_ref, wg_ref, wu_ref, wd_ref, y_ref):
    del be_ref
    xb = x_ref[...].astype(jnp.bfloat16)
    wg = wg_ref[0].astype(jnp.bfloat16)
    wu = wu_ref[0].astype(jnp.bfloat16)
    wd = wd_ref[0].astype(jnp.bfloat16)
    h = jnp.dot(xb, wg, preferred_element_type=jnp.float32)
    u = jnp.dot(xb, wu, preferred_element_type=jnp.float32)
    a = (h * (1.0 / (1.0 + jnp.exp(-h))) * u).astype(jnp.bfloat16)
    y_ref[...] = jnp.dot(a, wd, preferred_element_type=jnp.float32)


def _wsum_body(y0_ref, y1_ref, w0_ref, w1_ref, o_ref):
    o_ref[...] = w0_ref[...] * y0_ref[...] + w1_ref[...] * y1_ref[...]


def _sc_dims():
    try:
        info = plsc.get_sparse_core_info()
        return info.num_cores, info.num_subcores
    except Exception:
        return 2, 16


def _make_dispatch(nc, ns):
    nw = nc * ns
    tpw = T // nw
    mesh = plsc.VectorSubcoreMesh(core_axis_name="c", subcore_axis_name="s")

    @functools.partial(
        pl.kernel,
        out_type=jax.ShapeDtypeStruct((PAD_T, H), jnp.float32),
        mesh=mesh,
        scratch_types=[
            pltpu.VMEM((tpw,), jnp.int32),
            pltpu.VMEM((tpw,), jnp.int32),
            pltpu.VMEM((tpw, H), jnp.float32),
            pltpu.SemaphoreType.DMA,
        ],
    )
    def dispatch(x_hbm, s0_hbm, s1_hbm, out_hbm, i0_v, i1_v, rows_v, sem):
        wid = lax.axis_index("s") * nc + lax.axis_index("c")
        base = wid * tpw
        pltpu.sync_copy(s0_hbm.at[pl.ds(base, tpw)], i0_v)
        pltpu.sync_copy(s1_hbm.at[pl.ds(base, tpw)], i1_v)
        pltpu.sync_copy(x_hbm.at[pl.ds(base, tpw)], rows_v)
        pltpu.async_copy(rows_v, out_hbm.at[i0_v], sem).wait()
        pltpu.async_copy(rows_v, out_hbm.at[i1_v], sem).wait()

    return dispatch


def _make_combine(nc, ns):
    nw = nc * ns
    tpw = T // nw
    mesh = plsc.VectorSubcoreMesh(core_axis_name="c", subcore_axis_name="s")

    @functools.partial(
        pl.kernel,
        out_type=(jax.ShapeDtypeStruct((T, H), jnp.float32),
                  jax.ShapeDtypeStruct((T, H), jnp.float32)),
        mesh=mesh,
        scratch_types=[
            pltpu.VMEM((tpw,), jnp.int32),
            pltpu.VMEM((tpw, H), jnp.float32),
            pltpu.SemaphoreType.DMA,
        ],
    )
    def combine(y_hbm, g0_hbm, g1_hbm, y0_hbm, y1_hbm, i_v, rows_v, sem):
        wid = lax.axis_index("s") * nc + lax.axis_index("c")
        base = wid * tpw
        pltpu.sync_copy(g0_hbm.at[pl.ds(base, tpw)], i_v)
        pltpu.async_copy(y_hbm.at[i_v], rows_v, sem).wait()
        pltpu.sync_copy(rows_v, y0_hbm.at[pl.ds(base, tpw)])
        pltpu.sync_copy(g1_hbm.at[pl.ds(base, tpw)], i_v)
        pltpu.async_copy(y_hbm.at[i_v], rows_v, sem).wait()
        pltpu.sync_copy(rows_v, y1_hbm.at[pl.ds(base, tpw)])

    return combine


def _routing_call(x2, w_gate):
    col = lambda: jax.ShapeDtypeStruct((T, 1), jnp.float32)
    return pl.pallas_call(
        _routing_body,
        out_shape=(col(), col(), col(), col(), col(), col(),
                   jax.ShapeDtypeStruct((NB, 1), jnp.float32),
                   jax.ShapeDtypeStruct((8, 128), jnp.float32)),
    )(x2, w_gate)


def _mlp_call(be, xs, wg, wu, wd):
    grid_spec = pltpu.PrefetchScalarGridSpec(
        num_scalar_prefetch=1,
        grid=(NB,),
        in_specs=[
            pl.BlockSpec((BLK, H), lambda b, be: (b, 0)),
            pl.BlockSpec((1, H, FF), lambda b, be: (be[b], 0, 0)),
            pl.BlockSpec((1, H, FF), lambda b, be: (be[b], 0, 0)),
            pl.BlockSpec((1, FF, H), lambda b, be: (be[b], 0, 0)),
        ],
        out_specs=pl.BlockSpec((BLK, H), lambda b, be: (b, 0)),
    )
    return pl.pallas_call(
        _mlp_body,
        grid_spec=grid_spec,
        out_shape=jax.ShapeDtypeStruct((PAD_T, H), jnp.float32),
    )(be, xs, wg, wu, wd)


def _wsum_call(y0, y1, w0, w1):
    nblk = 8
    rb = T // nblk
    return pl.pallas_call(
        _wsum_body,
        grid=(nblk,),
        in_specs=[
            pl.BlockSpec((rb, H), lambda b: (b, 0)),
            pl.BlockSpec((rb, H), lambda b: (b, 0)),
            pl.BlockSpec((rb, 1), lambda b: (b, 0)),
            pl.BlockSpec((rb, 1), lambda b: (b, 0)),
        ],
        out_specs=pl.BlockSpec((rb, H), lambda b: (b, 0)),
        out_shape=jax.ShapeDtypeStruct((T, H), jnp.float32),
    )(y0, y1, w0, w1)


def kernel(x, W_gate, W_gate_proj, W_up_proj, W_down_proj):
    b, s, _ = x.shape
    x2 = x.reshape(T, H)
    s0f, s1f, g0f, g1f, w0, w1, bef, st = _routing_call(x2, W_gate)
    s0 = s0f.reshape(T).astype(jnp.int32)
    s1 = s1f.reshape(T).astype(jnp.int32)
    g0 = g0f.reshape(T).astype(jnp.int32)
    g1 = g1f.reshape(T).astype(jnp.int32)
    be = bef.reshape(NB).astype(jnp.int32)
    nc, ns = _sc_dims()
    xs = _make_dispatch(nc, ns)(x2, s0, s1)
    y = _mlp_call(be, xs, W_gate_proj, W_up_proj, W_down_proj)
    y0, y1 = _make_combine(nc, ns)(y, g0, g1)
    out2 = _wsum_call(y0, y1, w0, w1)
    output = out2.reshape(b, s, H)
    return output, st[0, 0], st[0, 1], st[0, 2], st[0, 3]


# speedup vs baseline: 1.0120x; 1.0120x over previous
import functools

import jax
import jax.numpy as jnp
from jax import lax
from jax.experimental import pallas as pl
from jax.experimental.pallas import tpu as pltpu
from jax.experimental.pallas import tpu_sc as plsc

H = 768
E = 8
FF = 768
T = 2048
BLK = 256
PAD_T = T * 2 + E * BLK
NB = PAD_T // BLK


def _routing_body(x_ref, wg_ref, s0_ref, s1_ref, g0_ref, g1_ref,
                  w0x_ref, w1x_ref, be_ref, st_ref):
    x = x_ref[...]
    wg = wg_ref[...]
    logits = jnp.dot(x, wg, preferred_element_type=jnp.float32)
    eidx = lax.broadcasted_iota(jnp.int32, (T, E), 1)
    m0 = jnp.max(logits, axis=1, keepdims=True)
    sel0 = jnp.min(jnp.where(logits == m0, eidx, E), axis=1, keepdims=True)
    masked = jnp.where(eidx == sel0, -jnp.inf, logits)
    m1 = jnp.max(masked, axis=1, keepdims=True)
    sel1 = jnp.min(jnp.where(masked == m1, eidx, E), axis=1, keepdims=True)
    ed = jnp.exp(m1 - m0)
    w1x_ref[...] = jnp.broadcast_to(ed / (1.0 + ed), (T, 16))
    w0x_ref[...] = jnp.broadcast_to(1.0 / (1.0 + ed), (T, 16))

    p = jnp.exp(logits - m0)
    probs = p / jnp.sum(p, axis=1, keepdims=True)
    usage = jnp.mean(probs, axis=0, keepdims=True)
    um = jnp.mean(usage)
    var = jnp.mean((usage - um) ** 2)
    balance = var * float(E)
    lane = lax.broadcasted_iota(jnp.int32, (1, E), 1)
    rsum = jnp.float32(0.0)
    for i in range(E):
        u_i = usage[0, i]
        less_i = jnp.sum(jnp.where(usage < u_i, 1.0, 0.0))
        tie_i = jnp.sum(jnp.where((usage == u_i) & (lane < i), 1.0, 0.0))
        rsum = rsum + u_i * (1.0 + less_i + tie_i)
    usum = jnp.sum(usage)
    gini = 2.0 * rsum / (float(E) * usum) - float(E + 1) / float(E)
    ent = -jnp.mean(jnp.sum(probs * jnp.log(probs + 1e-10), axis=1))

    oh0 = jnp.where(eidx == sel0, 1.0, 0.0)
    oh1 = jnp.where(eidx == sel1, 1.0, 0.0)
    oha = oh0 + oh1
    cb = 256
    r_i = lax.broadcasted_iota(jnp.int32, (cb, cb), 0)
    c_i = lax.broadcasted_iota(jnp.int32, (cb, cb), 1)
    ltri = jnp.where(r_i > c_i, 1.0, 0.0)

    def ex_cumsum(oh):
        run = jnp.zeros((1, E), jnp.float32)
        outs = []
        for b in range(T // cb):
            blk = lax.slice(oh, (b * cb, 0), ((b + 1) * cb, E))
            outs.append(run + jnp.dot(ltri, blk,
                                      preferred_element_type=jnp.float32))
            run = run + jnp.sum(blk, axis=0, keepdims=True)
        return jnp.concatenate(outs, axis=0), run

    rank0, c0 = ex_cumsum(oh0)
    rank1, c1 = ex_cumsum(oh1)
    ranka = rank0 + rank1
    ca = c0 + c1
    padc = jnp.floor((ca + float(BLK - 1)) / float(BLK)) * float(BLK)
    ii = lax.broadcasted_iota(jnp.int32, (E, E), 0)
    jj = lax.broadcasted_iota(jnp.int32, (E, E), 1)
    utri = jnp.where(ii < jj, 1.0, 0.0)
    off = jnp.dot(padc, utri, preferred_element_type=jnp.float32)
    s0_ref[...] = jnp.sum(oh0 * (ranka + off), axis=1,
                          keepdims=True).astype(jnp.int32)
    s1_ref[...] = jnp.sum(oh1 * (ranka + off), axis=1,
                          keepdims=True).astype(jnp.int32)
    g0_ref[...] = jnp.sum(oh0 * (rank0 + off), axis=1,
                          keepdims=True).astype(jnp.int32)
    g1_ref[...] = jnp.sum(oh1 * (rank1 + off), axis=1,
                          keepdims=True).astype(jnp.int32)

    bvals = lax.broadcasted_iota(jnp.int32, (NB, E), 0).astype(jnp.float32) * float(BLK)
    offnb = jnp.broadcast_to(off, (NB, E))
    cnt = jnp.sum(jnp.where(offnb <= bvals, 1.0, 0.0), axis=1, keepdims=True)
    be_ref[...] = jnp.clip(cnt - 1.0, 0.0, float(E - 1))

    sr = lax.broadcasted_iota(jnp.int32, (8, 128), 0)
    sc = lax.broadcasted_iota(jnp.int32, (8, 128), 1)
    st = jnp.where((sr == 0) & (sc == 0), balance, 0.0)
    st = st + jnp.where((sr == 0) & (sc == 1), var, 0.0)
    st = st + jnp.where((sr == 0) & (sc == 2), gini, 0.0)
    st_ref[...] = st + jnp.where((sr == 0) & (sc == 3), ent, 0.0)


def _mlp_body(be_ref, x_ref, wg_ref, wu_ref, wd_ref, y_ref):
    del be_ref
    xb = x_ref[...].astype(jnp.bfloat16)
    wg = wg_ref[0].astype(jnp.bfloat16)
    wu = wu_ref[0].astype(jnp.bfloat16)
    wd = wd_ref[0].astype(jnp.bfloat16)
    h = jnp.dot(xb, wg, preferred_element_type=jnp.float32)
    u = jnp.dot(xb, wu, preferred_element_type=jnp.float32)
    a = (h * (1.0 / (1.0 + jnp.exp(-h))) * u).astype(jnp.bfloat16)
    y_ref[...] = jnp.dot(a, wd, preferred_element_type=jnp.float32)


def _sc_dims():
    try:
        info = plsc.get_sparse_core_info()
        return info.num_cores, info.num_subcores
    except Exception:
        return 2, 16


def _make_dispatch(nc, ns):
    nw = nc * ns
    tpw = T // nw
    mesh = plsc.VectorSubcoreMesh(core_axis_name="c", subcore_axis_name="s")

    @functools.partial(
        pl.kernel,
        out_type=jax.ShapeDtypeStruct((PAD_T, H), jnp.float32),
        mesh=mesh,
        scratch_types=[
            pltpu.VMEM((tpw,), jnp.int32),
            pltpu.VMEM((tpw,), jnp.int32),
            pltpu.VMEM((tpw, H), jnp.float32),
            pltpu.SemaphoreType.DMA,
        ],
    )
    def dispatch(x_hbm, s0_hbm, s1_hbm, out_hbm, i0_v, i1_v, rows_v, sem):
        wid = lax.axis_index("s") * nc + lax.axis_index("c")
        base = wid * tpw
        pltpu.sync_copy(s0_hbm.at[pl.ds(base, tpw)], i0_v)
        pltpu.sync_copy(s1_hbm.at[pl.ds(base, tpw)], i1_v)
        pltpu.sync_copy(x_hbm.at[pl.ds(base, tpw)], rows_v)
        pltpu.async_copy(rows_v, out_hbm.at[i0_v], sem).wait()
        pltpu.async_copy(rows_v, out_hbm.at[i1_v], sem).wait()

    return dispatch


def _make_combine(nc, ns):
    nw = nc * ns
    tpw = T // nw
    sub = 32
    nsub = tpw // sub
    mesh = plsc.VectorSubcoreMesh(core_axis_name="c", subcore_axis_name="s")

    @functools.partial(
        pl.kernel,
        out_type=jax.ShapeDtypeStruct((T, H), jnp.float32),
        mesh=mesh,
        scratch_types=[
            pltpu.VMEM((sub,), jnp.int32),
            pltpu.VMEM((sub,), jnp.int32),
            pltpu.VMEM((sub, 16), jnp.float32),
            pltpu.VMEM((sub, 16), jnp.float32),
            pltpu.VMEM((sub, H), jnp.float32),
            pltpu.VMEM((sub, H), jnp.float32),
            pltpu.VMEM((sub, H), jnp.float32),
            pltpu.SemaphoreType.DMA,
        ],
    )
    def combine(y_hbm, g0_hbm, g1_hbm, w0x_hbm, w1x_hbm, out_hbm,
                i0_v, i1_v, w0_v, w1_v, r0_v, r1_v, o_v, sem):
        wid = lax.axis_index("s") * nc + lax.axis_index("c")
        for c in range(nsub):
            base = wid * tpw + c * sub
            pltpu.sync_copy(g0_hbm.at[pl.ds(base, sub)], i0_v)
            pltpu.sync_copy(g1_hbm.at[pl.ds(base, sub)], i1_v)
            pltpu.sync_copy(w0x_hbm.at[pl.ds(base, sub)], w0_v)
            pltpu.sync_copy(w1x_hbm.at[pl.ds(base, sub)], w1_v)
            d0 = pltpu.async_copy(y_hbm.at[i0_v], r0_v, sem)
            d1 = pltpu.async_copy(y_hbm.at[i1_v], r1_v, sem)
            d0.wait()
            d1.wait()

            def body(t, _):
                w0 = w0_v[t]
                w1 = w1_v[t]
                for j in range(H // 16):
                    sl = pl.ds(j * 16, 16)
                    o_v[t, sl] = r0_v[t, sl] * w0 + r1_v[t, sl] * w1
                return 0

            lax.fori_loop(0, sub, body, 0)
            pltpu.sync_copy(o_v, out_hbm.at[pl.ds(base, sub)])

    return combine


def _routing_call(x2, w_gate):
    icol = lambda: jax.ShapeDtypeStruct((T, 1), jnp.int32)
    wrow = lambda: jax.ShapeDtypeStruct((T, 16), jnp.float32)
    return pl.pallas_call(
        _routing_body,
        out_shape=(icol(), icol(), icol(), icol(), wrow(), wrow(),
                   jax.ShapeDtypeStruct((NB, 1), jnp.float32),
                   jax.ShapeDtypeStruct((8, 128), jnp.float32)),
    )(x2, w_gate)


def _mlp_call(be, xs, wg, wu, wd):
    grid_spec = pltpu.PrefetchScalarGridSpec(
        num_scalar_prefetch=1,
        grid=(NB,),
        in_specs=[
            pl.BlockSpec((BLK, H), lambda b, be: (b, 0)),
            pl.BlockSpec((1, H, FF), lambda b, be: (be[b], 0, 0)),
            pl.BlockSpec((1, H, FF), lambda b, be: (be[b], 0, 0)),
            pl.BlockSpec((1, FF, H), lambda b, be: (be[b], 0, 0)),
        ],
        out_specs=pl.BlockSpec((BLK, H), lambda b, be: (b, 0)),
    )
    return pl.pallas_call(
        _mlp_body,
        grid_spec=grid_spec,
        out_shape=jax.ShapeDtypeStruct((PAD_T, H), jnp.float32),
    )(be, xs, wg, wu, wd)


def kernel(x, W_gate, W_gate_proj, W_up_proj, W_down_proj):
    b, s, _ = x.shape
    x2 = x.reshape(T, H)
    s0f, s1f, g0f, g1f, w0x, w1x, bef, st = _routing_call(x2, W_gate)
    s0 = s0f.reshape(T)
    s1 = s1f.reshape(T)
    g0 = g0f.reshape(T)
    g1 = g1f.reshape(T)
    be = bef.reshape(NB).astype(jnp.int32)
    nc, ns = _sc_dims()
    xs = _make_dispatch(nc, ns)(x2, s0, s1)
    y = _mlp_call(be, xs, W_gate_proj, W_up_proj, W_down_proj)
    out2 = _make_combine(nc, ns)(y, g0, g1, w0x, w1x)
    output = out2.reshape(b, s, H)
    return output, st[0, 0], st[0, 1], st[0, 2], st[0, 3]
